# Initial kernel scaffold; baseline (speedup 1.0000x reference)
import jax
import jax.numpy as jnp
from jax.experimental import pallas as pl


def kernel(angles, tertiary, params, edge_index, subgraph):
    raise NotImplementedError("write your pallas kernel here")


# speedup vs baseline: 5.0716x; 5.0716x over previous
import functools

import numpy as np
import jax
import jax.numpy as jnp
from jax import lax
from jax.experimental import pallas as pl
from jax.experimental.pallas import tpu as pltpu
from jax.experimental.pallas import tpu_sc as plsc

SIZE = 512
HEADS = 8
ATT = 64
HA = HEADS * ATT
MIX = 10
KERN = 16
MAXD = 20.0
EDGE_F = KERN + 9
HID = 1024
N = 4096
E = 65536

FW = 32
NC = 2
NS = 16
HHALF = HEADS // NC
DH = HHALF * ATT
QW = DH + HHALF * FW
KVW = 2 * DH
ACCW = DH + HHALF * FW
RT = 512
CH = 16
EPT = E // NS
NPT = N // NS
INV_SQRT_ATT = 1.0 / 8.0

_f32 = jnp.float32


def _ln(x, g, b):
    mu = jnp.mean(x, axis=-1, keepdims=True)
    var = jnp.mean((x - mu) ** 2, axis=-1, keepdims=True)
    return (x - mu) * lax.rsqrt(var + 1e-5) * g + b


def _pre_body(af_ref, wpre_ref, bpre_ref, wenc_ref, benc_ref, o_ref):
    h = jnp.dot(af_ref[...], wpre_ref[...], preferred_element_type=_f32) + bpre_ref[...]
    o_ref[...] = jnp.dot(h, wenc_ref[...], preferred_element_type=_f32) + benc_ref[...]


def _tc_pre(af_p, wpre_p, bpre, wenc, benc):
    return pl.pallas_call(
        _pre_body,
        grid=(N // RT,),
        in_specs=[
            pl.BlockSpec((RT, 128), lambda i: (i, 0)),
            pl.BlockSpec((128, SIZE), lambda i: (0, 0)),
            pl.BlockSpec((1, SIZE), lambda i: (0, 0)),
            pl.BlockSpec((SIZE, SIZE), lambda i: (0, 0)),
            pl.BlockSpec((1, SIZE), lambda i: (0, 0)),
        ],
        out_specs=pl.BlockSpec((RT, SIZE), lambda i: (i, 0)),
        out_shape=jax.ShapeDtypeStruct((N, SIZE), _f32),
    )(af_p, wpre_p, bpre, wenc, benc)


def _ef_body(prd_ref, prs_ref, g1_ref, g2_ref, g3_ref, o_ref):
    prd = prd_ref[:, 0:16]
    prs = prs_ref[:, 0:16]
    diff = prd - prs
    mask = lax.broadcasted_iota(jnp.int32, (1, 16), 1) < 3
    d2 = jnp.sum(jnp.where(mask, diff * diff, 0.0), axis=1, keepdims=True)
    d = jnp.sqrt(d2)
    centers = lax.broadcasted_iota(jnp.int32, (1, KERN), 1).astype(_f32) * (MAXD / (KERN - 1))
    sigma = MAXD / KERN
    t = (d - centers) / sigma
    rbf = jnp.exp(-(t * t))
    t1 = jnp.dot(prd, g1_ref[...], preferred_element_type=_f32)
    t2 = jnp.dot(prs, g2_ref[...], preferred_element_type=_f32)
    rel = jnp.dot(t1 * t2, g3_ref[...], preferred_element_type=_f32)
    r = rbf.shape[0]
    o_ref[:, 0:KERN] = rbf
    o_ref[:, KERN:KERN + 9] = rel[:, 0:9]
    o_ref[:, KERN + 9:KERN + 10] = jnp.ones((r, 1), _f32)
    o_ref[:, KERN + 10:FW] = jnp.zeros((r, FW - EDGE_F - 1), _f32)


def _tc_ef(prd, prs, g1, g2, g3):
    rt = 2048
    return pl.pallas_call(
        _ef_body,
        grid=(E // rt,),
        in_specs=[
            pl.BlockSpec((rt, 128), lambda i: (i, 0)),
            pl.BlockSpec((rt, 128), lambda i: (i, 0)),
            pl.BlockSpec((16, 32), lambda i: (0, 0)),
            pl.BlockSpec((16, 32), lambda i: (0, 0)),
            pl.BlockSpec((32, 16), lambda i: (0, 0)),
        ],
        out_specs=pl.BlockSpec((rt, FW), lambda i: (i, 0)),
        out_shape=jax.ShapeDtypeStruct((E, FW), _f32),
    )(prd, prs, g1, g2, g3)


def _blkA_body(x_ref, g_ref, b_ref, wq_ref, bq_ref, wk_ref, bk_ref,
               wv_ref, bv_ref, wbd_ref, qq_ref, kv_ref):
    h = _ln(x_ref[...], g_ref[...], b_ref[...])
    qs = (jnp.dot(h, wq_ref[...], preferred_element_type=_f32) + bq_ref[...]) * INV_SQRT_ATT
    kn = jnp.dot(h, wk_ref[...], preferred_element_type=_f32) + bk_ref[...]
    vn = jnp.dot(h, wv_ref[...], preferred_element_type=_f32) + bv_ref[...]
    qek = jnp.dot(qs, wbd_ref[...], preferred_element_type=_f32)
    qq_ref[0, :, 0:DH] = qs[:, 0:DH]
    qq_ref[0, :, DH:QW] = qek[:, 0:HHALF * FW]
    qq_ref[1, :, 0:DH] = qs[:, DH:HA]
    qq_ref[1, :, DH:QW] = qek[:, HHALF * FW:2 * HHALF * FW]
    kv_ref[0, :, 0:DH] = kn[:, 0:DH]
    kv_ref[0, :, DH:KVW] = vn[:, 0:DH]
    kv_ref[1, :, 0:DH] = kn[:, DH:HA]
    kv_ref[1, :, DH:KVW] = vn[:, DH:HA]


def _tc_blkA(x, g, b, wq, bq, wk, bk, wv, bv, wbd):
    full = lambda shape: pl.BlockSpec(shape, lambda i: tuple(0 for _ in shape))
    return pl.pallas_call(
        _blkA_body,
        grid=(N // RT,),
        in_specs=[
            pl.BlockSpec((RT, SIZE), lambda i: (i, 0)),
            full((1, SIZE)), full((1, SIZE)),
            full((SIZE, HA)), full((1, HA)),
            full((SIZE, HA)), full((1, HA)),
            full((SIZE, HA)), full((1, HA)),
            full((HA, HEADS * FW)),
        ],
        out_specs=[
            pl.BlockSpec((NC, RT, QW), lambda i: (0, i, 0)),
            pl.BlockSpec((NC, RT, KVW), lambda i: (0, i, 0)),
        ],
        out_shape=[
            jax.ShapeDtypeStruct((NC, N, QW), _f32),
            jax.ShapeDtypeStruct((NC, N, KVW), _f32),
        ],
    )(x, g, b, wq, bq, wk, bk, wv, bv, wbd)


def _blkB_body(x_ref, acc_ref, wev_ref, zsel_ref, wo_ref, bo_ref,
               g2_ref, b2_ref, w1_ref, b1_ref, w2_ref, bb2_ref, w3_ref, b3_ref,
               o_ref):
    numv = jnp.concatenate([acc_ref[0, :, 0:DH], acc_ref[1, :, 0:DH]], axis=1)
    pmat = jnp.concatenate([acc_ref[0, :, DH:ACCW], acc_ref[1, :, DH:ACCW]], axis=1)
    numc = jnp.dot(pmat, wev_ref[...], preferred_element_type=_f32)
    zfull = jnp.dot(pmat, zsel_ref[...], preferred_element_type=_f32)
    o = (numv + numc) / (zfull + 1e-9)
    x2 = x_ref[...] + jnp.dot(o, wo_ref[...], preferred_element_type=_f32) + bo_ref[...]
    h2 = _ln(x2, g2_ref[...], b2_ref[...])
    m1 = jnp.maximum(jnp.dot(h2, w1_ref[...], preferred_element_type=_f32) + b1_ref[...], 0.0)
    m2 = jnp.maximum(jnp.dot(m1, w2_ref[...], preferred_element_type=_f32) + bb2_ref[...], 0.0)
    o_ref[...] = x2 + jnp.dot(m2, w3_ref[...], preferred_element_type=_f32) + b3_ref[...]


def _tc_blkB(x, acc, wev, zsel, wo, bo, g2, b2, w1, b1, w2, bb2, w3, b3):
    full = lambda shape: pl.BlockSpec(shape, lambda i: tuple(0 for _ in shape))
    return pl.pallas_call(
        _blkB_body,
        grid=(N // RT,),
        in_specs=[
            pl.BlockSpec((RT, SIZE), lambda i: (i, 0)),
            pl.BlockSpec((NC, RT, ACCW), lambda i: (0, i, 0)),
            full((HEADS * FW, HA)), full((HEADS * FW, HA)),
            full((HA, SIZE)), full((1, SIZE)),
            full((1, SIZE)), full((1, SIZE)),
            full((SIZE, HID)), full((1, HID)),
            full((HID, HID)), full((1, HID)),
            full((HID, SIZE)), full((1, SIZE)),
        ],
        out_specs=pl.BlockSpec((RT, SIZE), lambda i: (i, 0)),
        out_shape=jax.ShapeDtypeStruct((N, SIZE), _f32),
    )(x, acc, wev, zsel, wo, bo, g2, b2, w1, b1, w2, bb2, w3, b3)


def _head_body(enc_ref, ang_ref, wh_ref, bh_ref, o_ref):
    t = jnp.dot(enc_ref[...], wh_ref[...], preferred_element_type=_f32) + bh_ref[...]
    lg = t[:, 0:MIX]
    m = jnp.max(lg, axis=1, keepdims=True)
    ex = jnp.exp(lg - m)
    wts = ex / jnp.sum(ex, axis=1, keepdims=True)
    a0 = ang_ref[:, 0:1]
    a1 = ang_ref[:, 1:2]
    f0 = t[:, 40:50]
    f1 = t[:, 50:60]
    f2 = t[:, 60:70]
    mean1 = t[:, 20:30] + f0 * a0
    mean2 = t[:, 30:40] + f1 * a0 + f2 * a1
    conc = 0.1 + 1000.0 / (1.0 + jnp.exp(-t[:, 70:100]))
    o_ref[:, 0:10] = wts
    o_ref[:, 10:20] = t[:, 10:20]
    o_ref[:, 20:30] = mean1
    o_ref[:, 30:40] = mean2
    o_ref[:, 40:70] = t[:, 40:70]
    o_ref[:, 70:100] = conc
    o_ref[:, 100:128] = jnp.zeros_like(t[:, 100:128])


def _tc_head(enc, ang_p, wh, bh):
    return pl.pallas_call(
        _head_body,
        grid=(N // RT,),
        in_specs=[
            pl.BlockSpec((RT, SIZE), lambda i: (i, 0)),
            pl.BlockSpec((RT, 128), lambda i: (i, 0)),
            pl.BlockSpec((SIZE, 128), lambda i: (0, 0)),
            pl.BlockSpec((1, 128), lambda i: (0, 0)),
        ],
        out_specs=pl.BlockSpec((RT, 128), lambda i: (i, 0)),
        out_shape=jax.ShapeDtypeStruct((N, 128), _f32),
    )(enc, ang_p, wh, bh)


CPR = 128


def _prgather_body(pr_hbm, dst_hbm, src_hbm, prd_hbm, prs_hbm, idx_v, buf_v, sem):
    c = lax.axis_index("c")
    s = lax.axis_index("s")
    wid = s * NC + c
    nper = E // (NC * NS)
    base0 = wid * nper

    def body(j, carry):
        base = base0 + j * CPR
        pltpu.sync_copy(dst_hbm.at[pl.ds(base, CPR)], idx_v)
        pltpu.async_copy(pr_hbm.at[idx_v], buf_v, sem).wait()
        pltpu.sync_copy(buf_v, prd_hbm.at[pl.ds(base, CPR)])
        pltpu.sync_copy(src_hbm.at[pl.ds(base, CPR)], idx_v)
        pltpu.async_copy(pr_hbm.at[idx_v], buf_v, sem).wait()
        pltpu.sync_copy(buf_v, prs_hbm.at[pl.ds(base, CPR)])
        return carry

    lax.fori_loop(0, nper // CPR, body, 0)


@functools.cache
def _get_sc_prgather():
    return pl.kernel(
        _prgather_body,
        out_type=(
            jax.ShapeDtypeStruct((E, 128), _f32),
            jax.ShapeDtypeStruct((E, 128), _f32),
        ),
        mesh=plsc.VectorSubcoreMesh(core_axis_name="c", subcore_axis_name="s"),
        scratch_types=[
            pltpu.VMEM((CPR,), jnp.int32),
            pltpu.VMEM((CPR, 128), _f32),
            pltpu.SemaphoreType.DMA,
        ],
    )


def _sc_prgather(pr, dst, src):
    return _get_sc_prgather()(pr, dst, src)


_GDN = lax.GatherDimensionNumbers(
    offset_dims=(), collapsed_slice_dims=(0,), start_index_map=(0,))


def _perm16(t, idx):
    return lax.gather(t, idx[:, None], dimension_numbers=_GDN,
                      slice_sizes=(1,),
                      mode=lax.GatherScatterMode.PROMISE_IN_BOUNDS)


def _allsum16(t):
    io = lax.iota(jnp.int32, 16)
    for k in (8, 4, 2, 1):
        t = t + _perm16(t, io ^ k)
    return t


def _edge_body(qq_hbm, kv_hbm, ef_hbm, dst_hbm, src_hbm, bounds_hbm, out_hbm,
               idx_dq, idx_sq, qq_b, kv_b, ef_b, acc_t, vmem_b,
               sem0, sem1):
    c = lax.axis_index("c")
    s = lax.axis_index("s")
    cn = c * N
    io16 = lax.iota(jnp.int32, 16)

    zv = jnp.zeros((16,), _f32)

    def zrow(r, carry):
        for i in range(ACCW // 16):
            acc_t[pl.ds(r * ACCW + i * 16, 16)] = zv
        return carry

    lax.fori_loop(0, NPT, zrow, 0)

    pltpu.sync_copy(bounds_hbm, vmem_b)
    b_lo = vmem_b[pl.ds(0, 16)]
    b_hi = vmem_b[pl.ds(16, 16)]
    start_raw = jnp.sum(jnp.where(io16 == s, b_lo, 0))
    end_raw = (jnp.sum(jnp.where(io16 == s + 1, b_lo, 0)) +
               jnp.sum(jnp.where(io16 == s - 15, b_hi, 0)))
    start = (start_raw // CH) * CH
    nch = ((end_raw + CH - 1) // CH * CH - start) // CH

    def chunk(j, carry):
        base = start + j * CH
        pltpu.sync_copy(dst_hbm.at[pl.ds(base, CH)], idx_dq)
        pltpu.sync_copy(src_hbm.at[pl.ds(base, CH)], idx_sq)
        dstv = idx_dq[pl.ds(0, 16)]
        locv0 = dstv - s * NPT
        idx_dq[pl.ds(0, 16)] = dstv + cn
        idx_sq[pl.ds(0, 16)] = idx_sq[pl.ds(0, 16)] + cn
        pltpu.sync_copy(ef_hbm.at[pl.ds(base * FW, CH * FW)], ef_b)
        cp0 = pltpu.async_copy(qq_hbm.at[idx_dq], qq_b, sem0)
        cp1 = pltpu.async_copy(kv_hbm.at[idx_sq], kv_b, sem1)
        cp0.wait()
        cp1.wait()

        def accum(idxv, val):
            cur = plsc.load_gather(acc_t, [idxv])
            plsc.store_scatter(acc_t, [idxv], cur + val)

        def edge(e, ecarry):
            ev = jnp.full((16,), e, jnp.int32)
            lv = _perm16(locv0, ev)
            own = jnp.where((lv >= 0) & (lv < NPT), 1.0, 0.0)
            rbase = jnp.clip(lv, 0, NPT - 1) * ACCW + io16
            ef0 = ef_b[pl.ds(e * FW, 16)]
            ef1 = ef_b[pl.ds(e * FW + 16, 16)]
            for h in range(HHALF):
                qb = h * ATT
                prod = (qq_b[e, pl.ds(qb, 16)] * kv_b[e, pl.ds(qb, 16)]
                        + qq_b[e, pl.ds(qb + 16, 16)] * kv_b[e, pl.ds(qb + 16, 16)]
                        + qq_b[e, pl.ds(qb + 32, 16)] * kv_b[e, pl.ds(qb + 32, 16)]
                        + qq_b[e, pl.ds(qb + 48, 16)] * kv_b[e, pl.ds(qb + 48, 16)])
                p2 = (qq_b[e, pl.ds(DH + h * FW, 16)] * ef0
                      + qq_b[e, pl.ds(DH + h * FW + 16, 16)] * ef1)
                exv = jnp.exp(_allsum16(prod + p2)) * own
                for i in range(ATT // 16):
                    accum(rbase + (qb + i * 16),
                          exv * kv_b[e, pl.ds(DH + qb + i * 16, 16)])
                accum(rbase + (DH + h * FW), exv * ef0)
                accum(rbase + (DH + h * FW + 16), exv * ef1)
            return ecarry

        lax.fori_loop(0, CH, edge, 0)
        return carry

    lax.fori_loop(0, nch, chunk, 0)
    pltpu.sync_copy(acc_t, out_hbm.at[pl.ds((cn + s * NPT) * ACCW, NPT * ACCW)])


@functools.cache
def _get_sc_edge():
    return pl.kernel(
        _edge_body,
        out_type=jax.ShapeDtypeStruct((NC * N * ACCW,), _f32),
        mesh=plsc.VectorSubcoreMesh(core_axis_name="c", subcore_axis_name="s"),
        compiler_params=pltpu.CompilerParams(needs_layout_passes=False),
        scratch_types=[
            pltpu.VMEM((CH,), jnp.int32),
            pltpu.VMEM((CH,), jnp.int32),
            pltpu.VMEM((CH, QW), _f32),
            pltpu.VMEM((CH, KVW), _f32),
            pltpu.VMEM((CH * FW,), _f32),
            pltpu.VMEM((NPT * ACCW,), _f32),
            pltpu.VMEM((32,), jnp.int32),
            pltpu.SemaphoreType.DMA,
            pltpu.SemaphoreType.DMA,
        ],
    )


def _sc_edge(qq, kv, ef, dst, src, bounds):
    return _get_sc_edge()(qq, kv, ef, dst, src, bounds)


def _orient(tertiary):
    pos = tertiary[:, 1]
    nxt = jnp.roll(pos, -1, axis=0)
    prv = jnp.roll(pos, 1, axis=0)
    a = nxt - pos
    a = a / (jnp.linalg.norm(a, axis=-1, keepdims=True) + 1e-8)
    cc = pos - prv
    cc = cc / (jnp.linalg.norm(cc, axis=-1, keepdims=True) + 1e-8)
    nvec = jnp.cross(a, cc)
    nvec = nvec / (jnp.linalg.norm(nvec, axis=-1, keepdims=True) + 1e-8)
    m = jnp.cross(nvec, a)
    return jnp.stack([a, m, nvec], axis=1)


def _sel_mats():
    g1 = np.zeros((16, 32), np.float32)
    g2 = np.zeros((16, 32), np.float32)
    g3 = np.zeros((32, 16), np.float32)
    for i in range(3):
        for j in range(3):
            for k in range(3):
                b = 9 * i + 3 * j + k
                g1[3 + 3 * i + j, b] = 1.0
                g2[3 + 3 * i + k, b] = 1.0
                g3[b, 3 * j + k] = 1.0
    return g1, g2, g3[:, :16]


_G1, _G2, _G3 = _sel_mats()


def _block_weights(blk):
    wek = blk['Wek']
    bek = blk['bek']
    wev = blk['Wev']
    wbd = jnp.zeros((HA, HEADS * FW), _f32)
    wevbd = jnp.zeros((HEADS * FW, HA), _f32)
    zsel = jnp.zeros((HEADS * FW, HA), _f32)
    for h in range(HEADS):
        cs = slice(h * ATT, (h + 1) * ATT)
        wbd = wbd.at[cs, h * FW:h * FW + EDGE_F].set(wek[:, cs].T)
        wbd = wbd.at[cs, h * FW + EDGE_F].set(bek[cs])
        wevbd = wevbd.at[h * FW:h * FW + EDGE_F, cs].set(wev[:, cs])
        zsel = zsel.at[h * FW + EDGE_F, cs].set(1.0)
    return wbd, wevbd, zsel


def kernel(angles, tertiary, params, edge_index, subgraph):
    p = params
    row = lambda v: v.reshape(1, -1)

    perm = jnp.argsort(edge_index[0])
    dst = edge_index[0][perm]
    src = edge_index[1][perm]
    boundaries = jnp.arange(0, N + 1, NPT, dtype=jnp.int32)
    bounds = jnp.searchsorted(dst, boundaries).astype(jnp.int32)
    bounds_p = jnp.zeros((32,), jnp.int32).at[0:NS + 1].set(bounds)

    prev = jnp.roll(angles, 1, axis=0).at[0].set(0.0)
    afeat = jnp.concatenate([jnp.sin(prev), jnp.cos(prev)], axis=1)
    af_p = jnp.pad(afeat, ((0, 0), (0, 122)))
    wpre_p = jnp.pad(p['W_pre'], ((0, 122), (0, 0)))

    pos = tertiary[:, 1]
    rot = _orient(tertiary)
    pr = jnp.concatenate([pos, rot.reshape(N, 9), jnp.zeros((N, 116), _f32)], axis=1)

    x = _tc_pre(af_p, wpre_p, row(p['b_pre']), p['W_enc'], row(p['b_enc']))

    prd, prs = _sc_prgather(pr, dst, src)
    ef = _tc_ef(prd, prs, _G1, _G2, _G3)
    ef_flat = ef.reshape(-1)

    for blk in p['blocks']:
        wbd, wevbd, zsel = _block_weights(blk)
        qq, kv = _tc_blkA(
            x, row(blk['ln1_g']), row(blk['ln1_b']),
            blk['Wq'], row(blk['bq']),
            blk['Wk'], row(blk['bk']),
            blk['Wv'], row(blk['bv'] + blk['bev']),
            wbd)
        acc = _sc_edge(qq.reshape(NC * N, QW), kv.reshape(NC * N, KVW),
                       ef_flat, dst, src, bounds_p)
        x = _tc_blkB(
            x, acc.reshape(NC, N, ACCW), wevbd, zsel,
            blk['Wo'], row(blk['bo']),
            row(blk['ln2_g']), row(blk['ln2_b']),
            blk['W1'], row(blk['b1']),
            blk['W2'], row(blk['b2']),
            blk['W3'], row(blk['b3']))

    wh = jnp.zeros((SIZE, 128), _f32)
    wh = wh.at[:, 0:10].set(p['W_wts'])
    wh = wh.at[:, 10:40].set(p['W_mean'])
    wh = wh.at[:, 40:70].set(p['W_fac'])
    wh = wh.at[:, 70:100].set(p['W_conc'])
    bh = jnp.zeros((128,), _f32)
    bh = bh.at[0:10].set(p['b_wts'])
    bh = bh.at[10:40].set(p['b_mean'])
    bh = bh.at[40:70].set(p['b_fac'])
    bh = bh.at[70:100].set(p['b_conc'])
    ang_p = jnp.pad(angles, ((0, 0), (0, 125)))
    out = _tc_head(x, ang_p, wh, row(bh))

    wts = out[:, 0:10]
    mean = out[:, 10:40].reshape(N, 3, MIX)
    factor = out[:, 40:70].reshape(N, 3, MIX)
    conc = out[:, 70:100].reshape(N, 3, MIX)
    return wts, mean, conc, factor


# speedup vs baseline: 6.1787x; 1.2183x over previous
import functools

import numpy as np
import jax
import jax.numpy as jnp
from jax import lax
from jax.experimental import pallas as pl
from jax.experimental.pallas import tpu as pltpu
from jax.experimental.pallas import tpu_sc as plsc

SIZE = 512
HEADS = 8
ATT = 64
HA = HEADS * ATT
MIX = 10
KERN = 16
MAXD = 20.0
EDGE_F = KERN + 9
HID = 1024
N = 4096
E = 65536

FW = 32
NC = 2
NS = 16
HHALF = HEADS // NC
DH = HHALF * ATT
QW = DH + HHALF * FW
KVW = 2 * DH
ACCW = DH + HHALF * FW
RT = 512
CH = 16
EPT = E // NS
NPT = N // NS
INV_SQRT_ATT = 1.0 / 8.0

_f32 = jnp.float32


def _ln(x, g, b):
    mu = jnp.mean(x, axis=-1, keepdims=True)
    var = jnp.mean((x - mu) ** 2, axis=-1, keepdims=True)
    return (x - mu) * lax.rsqrt(var + 1e-5) * g + b


def _pre_body(af_ref, wpre_ref, bpre_ref, wenc_ref, benc_ref, o_ref):
    h = jnp.dot(af_ref[...], wpre_ref[...], preferred_element_type=_f32) + bpre_ref[...]
    o_ref[...] = jnp.dot(h, wenc_ref[...], preferred_element_type=_f32) + benc_ref[...]


def _tc_pre(af_p, wpre_p, bpre, wenc, benc):
    return pl.pallas_call(
        _pre_body,
        grid=(N // RT,),
        in_specs=[
            pl.BlockSpec((RT, 128), lambda i: (i, 0)),
            pl.BlockSpec((128, SIZE), lambda i: (0, 0)),
            pl.BlockSpec((1, SIZE), lambda i: (0, 0)),
            pl.BlockSpec((SIZE, SIZE), lambda i: (0, 0)),
            pl.BlockSpec((1, SIZE), lambda i: (0, 0)),
        ],
        out_specs=pl.BlockSpec((RT, SIZE), lambda i: (i, 0)),
        out_shape=jax.ShapeDtypeStruct((N, SIZE), _f32),
    )(af_p, wpre_p, bpre, wenc, benc)


def _ef_body(prd_ref, prs_ref, g1_ref, g2_ref, g3_ref, o_ref):
    prd = prd_ref[:, 0:16]
    prs = prs_ref[:, 0:16]
    diff = prd - prs
    mask = lax.broadcasted_iota(jnp.int32, (1, 16), 1) < 3
    d2 = jnp.sum(jnp.where(mask, diff * diff, 0.0), axis=1, keepdims=True)
    d = jnp.sqrt(d2)
    centers = lax.broadcasted_iota(jnp.int32, (1, KERN), 1).astype(_f32) * (MAXD / (KERN - 1))
    sigma = MAXD / KERN
    t = (d - centers) / sigma
    rbf = jnp.exp(-(t * t))
    t1 = jnp.dot(prd, g1_ref[...], preferred_element_type=_f32)
    t2 = jnp.dot(prs, g2_ref[...], preferred_element_type=_f32)
    rel = jnp.dot(t1 * t2, g3_ref[...], preferred_element_type=_f32)
    r = rbf.shape[0]
    o_ref[:, 0:KERN] = rbf
    o_ref[:, KERN:KERN + 9] = rel[:, 0:9]
    o_ref[:, KERN + 9:KERN + 10] = jnp.ones((r, 1), _f32)
    o_ref[:, KERN + 10:FW] = jnp.zeros((r, FW - EDGE_F - 1), _f32)


def _tc_ef(prd, prs, g1, g2, g3):
    rt = 2048
    return pl.pallas_call(
        _ef_body,
        grid=(E // rt,),
        in_specs=[
            pl.BlockSpec((rt, 128), lambda i: (i, 0)),
            pl.BlockSpec((rt, 128), lambda i: (i, 0)),
            pl.BlockSpec((16, 32), lambda i: (0, 0)),
            pl.BlockSpec((16, 32), lambda i: (0, 0)),
            pl.BlockSpec((32, 16), lambda i: (0, 0)),
        ],
        out_specs=pl.BlockSpec((rt, FW), lambda i: (i, 0)),
        out_shape=jax.ShapeDtypeStruct((E, FW), _f32),
    )(prd, prs, g1, g2, g3)


def _blkA_body(x_ref, g_ref, b_ref, wq_ref, bq_ref, wk_ref, bk_ref,
               wv_ref, bv_ref, wbd_ref, qq_ref, kv_ref):
    h = _ln(x_ref[...], g_ref[...], b_ref[...])
    qs = (jnp.dot(h, wq_ref[...], preferred_element_type=_f32) + bq_ref[...]) * INV_SQRT_ATT
    kn = jnp.dot(h, wk_ref[...], preferred_element_type=_f32) + bk_ref[...]
    vn = jnp.dot(h, wv_ref[...], preferred_element_type=_f32) + bv_ref[...]
    qek = jnp.dot(qs, wbd_ref[...], preferred_element_type=_f32)
    qq_ref[0, :, 0:DH] = qs[:, 0:DH]
    qq_ref[0, :, DH:QW] = qek[:, 0:HHALF * FW]
    qq_ref[1, :, 0:DH] = qs[:, DH:HA]
    qq_ref[1, :, DH:QW] = qek[:, HHALF * FW:2 * HHALF * FW]
    kv_ref[0, :, 0:DH] = kn[:, 0:DH]
    kv_ref[0, :, DH:KVW] = vn[:, 0:DH]
    kv_ref[1, :, 0:DH] = kn[:, DH:HA]
    kv_ref[1, :, DH:KVW] = vn[:, DH:HA]


def _tc_blkA(x, g, b, wq, bq, wk, bk, wv, bv, wbd):
    full = lambda shape: pl.BlockSpec(shape, lambda i: tuple(0 for _ in shape))
    return pl.pallas_call(
        _blkA_body,
        grid=(N // RT,),
        in_specs=[
            pl.BlockSpec((RT, SIZE), lambda i: (i, 0)),
            full((1, SIZE)), full((1, SIZE)),
            full((SIZE, HA)), full((1, HA)),
            full((SIZE, HA)), full((1, HA)),
            full((SIZE, HA)), full((1, HA)),
            full((HA, HEADS * FW)),
        ],
        out_specs=[
            pl.BlockSpec((NC, RT, QW), lambda i: (0, i, 0)),
            pl.BlockSpec((NC, RT, KVW), lambda i: (0, i, 0)),
        ],
        out_shape=[
            jax.ShapeDtypeStruct((NC, N, QW), _f32),
            jax.ShapeDtypeStruct((NC, N, KVW), _f32),
        ],
    )(x, g, b, wq, bq, wk, bk, wv, bv, wbd)


def _blkB_body(x_ref, acc_ref, wev_ref, zsel_ref, wo_ref, bo_ref,
               g2_ref, b2_ref, w1_ref, b1_ref, w2_ref, bb2_ref, w3_ref, b3_ref,
               o_ref):
    numv = jnp.concatenate([acc_ref[0, :, 0:DH], acc_ref[1, :, 0:DH]], axis=1)
    pmat = jnp.concatenate([acc_ref[0, :, DH:ACCW], acc_ref[1, :, DH:ACCW]], axis=1)
    numc = jnp.dot(pmat, wev_ref[...], preferred_element_type=_f32)
    zfull = jnp.dot(pmat, zsel_ref[...], preferred_element_type=_f32)
    o = (numv + numc) / (zfull + 1e-9)
    x2 = x_ref[...] + jnp.dot(o, wo_ref[...], preferred_element_type=_f32) + bo_ref[...]
    h2 = _ln(x2, g2_ref[...], b2_ref[...])
    m1 = jnp.maximum(jnp.dot(h2, w1_ref[...], preferred_element_type=_f32) + b1_ref[...], 0.0)
    m2 = jnp.maximum(jnp.dot(m1, w2_ref[...], preferred_element_type=_f32) + bb2_ref[...], 0.0)
    o_ref[...] = x2 + jnp.dot(m2, w3_ref[...], preferred_element_type=_f32) + b3_ref[...]


def _tc_blkB(x, acc, wev, zsel, wo, bo, g2, b2, w1, b1, w2, bb2, w3, b3):
    full = lambda shape: pl.BlockSpec(shape, lambda i: tuple(0 for _ in shape))
    return pl.pallas_call(
        _blkB_body,
        grid=(N // RT,),
        in_specs=[
            pl.BlockSpec((RT, SIZE), lambda i: (i, 0)),
            pl.BlockSpec((NC, RT, ACCW), lambda i: (0, i, 0)),
            full((HEADS * FW, HA)), full((HEADS * FW, HA)),
            full((HA, SIZE)), full((1, SIZE)),
            full((1, SIZE)), full((1, SIZE)),
            full((SIZE, HID)), full((1, HID)),
            full((HID, HID)), full((1, HID)),
            full((HID, SIZE)), full((1, SIZE)),
        ],
        out_specs=pl.BlockSpec((RT, SIZE), lambda i: (i, 0)),
        out_shape=jax.ShapeDtypeStruct((N, SIZE), _f32),
    )(x, acc, wev, zsel, wo, bo, g2, b2, w1, b1, w2, bb2, w3, b3)


def _head_body(enc_ref, ang_ref, wh_ref, bh_ref, o_ref):
    t = jnp.dot(enc_ref[...], wh_ref[...], preferred_element_type=_f32) + bh_ref[...]
    lg = t[:, 0:MIX]
    m = jnp.max(lg, axis=1, keepdims=True)
    ex = jnp.exp(lg - m)
    wts = ex / jnp.sum(ex, axis=1, keepdims=True)
    a0 = ang_ref[:, 0:1]
    a1 = ang_ref[:, 1:2]
    f0 = t[:, 40:50]
    f1 = t[:, 50:60]
    f2 = t[:, 60:70]
    mean1 = t[:, 20:30] + f0 * a0
    mean2 = t[:, 30:40] + f1 * a0 + f2 * a1
    conc = 0.1 + 1000.0 / (1.0 + jnp.exp(-t[:, 70:100]))
    o_ref[:, 0:10] = wts
    o_ref[:, 10:20] = t[:, 10:20]
    o_ref[:, 20:30] = mean1
    o_ref[:, 30:40] = mean2
    o_ref[:, 40:70] = t[:, 40:70]
    o_ref[:, 70:100] = conc
    o_ref[:, 100:128] = jnp.zeros_like(t[:, 100:128])


def _tc_head(enc, ang_p, wh, bh):
    return pl.pallas_call(
        _head_body,
        grid=(N // RT,),
        in_specs=[
            pl.BlockSpec((RT, SIZE), lambda i: (i, 0)),
            pl.BlockSpec((RT, 128), lambda i: (i, 0)),
            pl.BlockSpec((SIZE, 128), lambda i: (0, 0)),
            pl.BlockSpec((1, 128), lambda i: (0, 0)),
        ],
        out_specs=pl.BlockSpec((RT, 128), lambda i: (i, 0)),
        out_shape=jax.ShapeDtypeStruct((N, 128), _f32),
    )(enc, ang_p, wh, bh)


CPR = 128


def _prgather_body(pr_hbm, dst_hbm, src_hbm, prd_hbm, prs_hbm, idx_v, buf_v, sem):
    c = lax.axis_index("c")
    s = lax.axis_index("s")
    wid = s * NC + c
    nper = E // (NC * NS)
    base0 = wid * nper

    def body(j, carry):
        base = base0 + j * CPR
        pltpu.sync_copy(dst_hbm.at[pl.ds(base, CPR)], idx_v)
        pltpu.async_copy(pr_hbm.at[idx_v], buf_v, sem).wait()
        pltpu.sync_copy(buf_v, prd_hbm.at[pl.ds(base, CPR)])
        pltpu.sync_copy(src_hbm.at[pl.ds(base, CPR)], idx_v)
        pltpu.async_copy(pr_hbm.at[idx_v], buf_v, sem).wait()
        pltpu.sync_copy(buf_v, prs_hbm.at[pl.ds(base, CPR)])
        return carry

    lax.fori_loop(0, nper // CPR, body, 0)


@functools.cache
def _get_sc_prgather():
    return pl.kernel(
        _prgather_body,
        out_type=(
            jax.ShapeDtypeStruct((E, 128), _f32),
            jax.ShapeDtypeStruct((E, 128), _f32),
        ),
        mesh=plsc.VectorSubcoreMesh(core_axis_name="c", subcore_axis_name="s"),
        scratch_types=[
            pltpu.VMEM((CPR,), jnp.int32),
            pltpu.VMEM((CPR, 128), _f32),
            pltpu.SemaphoreType.DMA,
        ],
    )


def _sc_prgather(pr, dst, src):
    return _get_sc_prgather()(pr, dst, src)


_GDN = lax.GatherDimensionNumbers(
    offset_dims=(), collapsed_slice_dims=(0,), start_index_map=(0,))


def _perm16(t, idx):
    return lax.gather(t, idx[:, None], dimension_numbers=_GDN,
                      slice_sizes=(1,),
                      mode=lax.GatherScatterMode.PROMISE_IN_BOUNDS)


def _allsum16(t):
    io = lax.iota(jnp.int32, 16)
    for k in (8, 4, 2, 1):
        t = t + _perm16(t, io ^ k)
    return t


SCN = 16
SCB = SCN * CH


def _edge_body(qq_hbm, kv_hbm, ef_hbm, dst_hbm, src_hbm, bounds_hbm, out_hbm,
               idx_dq, idx_sq, qq_b, kv_b, ef_b, acc_t, vmem_b, sdst, ssrc,
               sem0, sem1):
    c = lax.axis_index("c")
    s = lax.axis_index("s")
    cn = c * N
    io16 = lax.iota(jnp.int32, 16)

    zv = jnp.zeros((16,), _f32)

    def zrow(r, carry):
        for i in range(ACCW // 16):
            acc_t[pl.ds(r * ACCW + i * 16, 16)] = zv
        return carry

    lax.fori_loop(0, NPT, zrow, 0)

    pltpu.sync_copy(bounds_hbm, vmem_b)
    b_lo = vmem_b[pl.ds(0, 16)]
    b_hi = vmem_b[pl.ds(16, 16)]
    start_raw = jnp.sum(jnp.where(io16 == s, b_lo, 0))
    end_raw = (jnp.sum(jnp.where(io16 == s + 1, b_lo, 0)) +
               jnp.sum(jnp.where(io16 == s - 15, b_hi, 0)))
    start = (start_raw // CH) * CH
    nch = ((end_raw + CH - 1) // CH * CH - start) // CH

    def chunk(j, carry):
        base = start + j * CH

        @pl.when(lax.rem(j, SCN) == 0)
        def _():
            sb = base
            pltpu.sync_copy(dst_hbm.at[pl.ds(sb, SCB)], sdst)
            pltpu.sync_copy(src_hbm.at[pl.ds(sb, SCB)], ssrc)

        soff = lax.rem(j, SCN) * CH
        dstv = sdst[pl.ds(soff, 16)]
        locv0 = dstv - s * NPT
        idx_dq[pl.ds(0, 16)] = dstv + cn
        idx_sq[pl.ds(0, 16)] = ssrc[pl.ds(soff, 16)] + cn
        pltpu.sync_copy(ef_hbm.at[pl.ds(base * FW, CH * FW)], ef_b)
        cp0 = pltpu.async_copy(qq_hbm.at[idx_dq], qq_b, sem0)
        cp1 = pltpu.async_copy(kv_hbm.at[idx_sq], kv_b, sem1)
        cp0.wait()
        cp1.wait()

        def edge(e, ecarry):
            loc = jnp.sum(jnp.where(io16 == e, locv0, 0))

            @pl.when((loc >= 0) & (loc < NPT))
            def _():
                rb = loc * ACCW
                ef0 = ef_b[pl.ds(e * FW, 16)]
                ef1 = ef_b[pl.ds(e * FW + 16, 16)]
                for h in range(HHALF):
                    qb = h * ATT
                    prod = (qq_b[e, pl.ds(qb, 16)] * kv_b[e, pl.ds(qb, 16)]
                            + qq_b[e, pl.ds(qb + 16, 16)] * kv_b[e, pl.ds(qb + 16, 16)]
                            + qq_b[e, pl.ds(qb + 32, 16)] * kv_b[e, pl.ds(qb + 32, 16)]
                            + qq_b[e, pl.ds(qb + 48, 16)] * kv_b[e, pl.ds(qb + 48, 16)])
                    p2 = (qq_b[e, pl.ds(DH + h * FW, 16)] * ef0
                          + qq_b[e, pl.ds(DH + h * FW + 16, 16)] * ef1)
                    exv = jnp.exp(_allsum16(prod + p2))
                    for i in range(ATT // 16):
                        plsc.addupdate(
                            acc_t.at[pl.ds(rb + qb + i * 16, 16)],
                            exv * kv_b[e, pl.ds(DH + qb + i * 16, 16)])
                    plsc.addupdate(acc_t.at[pl.ds(rb + DH + h * FW, 16)],
                                   exv * ef0)
                    plsc.addupdate(acc_t.at[pl.ds(rb + DH + h * FW + 16, 16)],
                                   exv * ef1)

            return ecarry

        lax.fori_loop(0, CH, edge, 0)
        return carry

    lax.fori_loop(0, nch, chunk, 0)
    pltpu.sync_copy(acc_t, out_hbm.at[pl.ds((cn + s * NPT) * ACCW, NPT * ACCW)])


@functools.cache
def _get_sc_edge():
    return pl.kernel(
        _edge_body,
        out_type=jax.ShapeDtypeStruct((NC * N * ACCW,), _f32),
        mesh=plsc.VectorSubcoreMesh(core_axis_name="c", subcore_axis_name="s"),
        compiler_params=pltpu.CompilerParams(needs_layout_passes=False),
        scratch_types=[
            pltpu.VMEM((CH,), jnp.int32),
            pltpu.VMEM((CH,), jnp.int32),
            pltpu.VMEM((CH, QW), _f32),
            pltpu.VMEM((CH, KVW), _f32),
            pltpu.VMEM((CH * FW,), _f32),
            pltpu.VMEM((NPT * ACCW,), _f32),
            pltpu.VMEM((32,), jnp.int32),
            pltpu.VMEM((SCB,), jnp.int32),
            pltpu.VMEM((SCB,), jnp.int32),
            pltpu.SemaphoreType.DMA,
            pltpu.SemaphoreType.DMA,
        ],
    )


def _sc_edge(qq, kv, ef, dst, src, bounds):
    return _get_sc_edge()(qq, kv, ef, dst, src, bounds)


def _orient(tertiary):
    pos = tertiary[:, 1]
    nxt = jnp.roll(pos, -1, axis=0)
    prv = jnp.roll(pos, 1, axis=0)
    a = nxt - pos
    a = a / (jnp.linalg.norm(a, axis=-1, keepdims=True) + 1e-8)
    cc = pos - prv
    cc = cc / (jnp.linalg.norm(cc, axis=-1, keepdims=True) + 1e-8)
    nvec = jnp.cross(a, cc)
    nvec = nvec / (jnp.linalg.norm(nvec, axis=-1, keepdims=True) + 1e-8)
    m = jnp.cross(nvec, a)
    return jnp.stack([a, m, nvec], axis=1)


def _sel_mats():
    g1 = np.zeros((16, 32), np.float32)
    g2 = np.zeros((16, 32), np.float32)
    g3 = np.zeros((32, 16), np.float32)
    for i in range(3):
        for j in range(3):
            for k in range(3):
                b = 9 * i + 3 * j + k
                g1[3 + 3 * i + j, b] = 1.0
                g2[3 + 3 * i + k, b] = 1.0
                g3[b, 3 * j + k] = 1.0
    return g1, g2, g3[:, :16]


_G1, _G2, _G3 = _sel_mats()


def _block_weights(blk):
    wek = blk['Wek']
    bek = blk['bek']
    wev = blk['Wev']
    wbd = jnp.zeros((HA, HEADS * FW), _f32)
    wevbd = jnp.zeros((HEADS * FW, HA), _f32)
    zsel = jnp.zeros((HEADS * FW, HA), _f32)
    for h in range(HEADS):
        cs = slice(h * ATT, (h + 1) * ATT)
        wbd = wbd.at[cs, h * FW:h * FW + EDGE_F].set(wek[:, cs].T)
        wbd = wbd.at[cs, h * FW + EDGE_F].set(bek[cs])
        wevbd = wevbd.at[h * FW:h * FW + EDGE_F, cs].set(wev[:, cs])
        zsel = zsel.at[h * FW + EDGE_F, cs].set(1.0)
    return wbd, wevbd, zsel


def kernel(angles, tertiary, params, edge_index, subgraph):
    p = params
    row = lambda v: v.reshape(1, -1)

    perm = jnp.argsort(edge_index[0])
    dst = edge_index[0][perm]
    src = edge_index[1][perm]
    boundaries = jnp.arange(0, N + 1, NPT, dtype=jnp.int32)
    bounds = jnp.searchsorted(dst, boundaries).astype(jnp.int32)
    bounds_p = jnp.zeros((32,), jnp.int32).at[0:NS + 1].set(bounds)
    dst_p = jnp.concatenate([dst, jnp.zeros((SCB,), jnp.int32)])
    src_p = jnp.concatenate([src, jnp.zeros((SCB,), jnp.int32)])

    prev = jnp.roll(angles, 1, axis=0).at[0].set(0.0)
    afeat = jnp.concatenate([jnp.sin(prev), jnp.cos(prev)], axis=1)
    af_p = jnp.pad(afeat, ((0, 0), (0, 122)))
    wpre_p = jnp.pad(p['W_pre'], ((0, 122), (0, 0)))

    pos = tertiary[:, 1]
    rot = _orient(tertiary)
    pr = jnp.concatenate([pos, rot.reshape(N, 9), jnp.zeros((N, 116), _f32)], axis=1)

    x = _tc_pre(af_p, wpre_p, row(p['b_pre']), p['W_enc'], row(p['b_enc']))

    prd, prs = _sc_prgather(pr, dst, src)
    ef = _tc_ef(prd, prs, _G1, _G2, _G3)
    ef_flat = ef.reshape(-1)

    for blk in p['blocks']:
        wbd, wevbd, zsel = _block_weights(blk)
        qq, kv = _tc_blkA(
            x, row(blk['ln1_g']), row(blk['ln1_b']),
            blk['Wq'], row(blk['bq']),
            blk['Wk'], row(blk['bk']),
            blk['Wv'], row(blk['bv'] + blk['bev']),
            wbd)
        acc = _sc_edge(qq.reshape(NC * N, QW), kv.reshape(NC * N, KVW),
                       ef_flat, dst_p, src_p, bounds_p)
        x = _tc_blkB(
            x, acc.reshape(NC, N, ACCW), wevbd, zsel,
            blk['Wo'], row(blk['bo']),
            row(blk['ln2_g']), row(blk['ln2_b']),
            blk['W1'], row(blk['b1']),
            blk['W2'], row(blk['b2']),
            blk['W3'], row(blk['b3']))

    wh = jnp.zeros((SIZE, 128), _f32)
    wh = wh.at[:, 0:10].set(p['W_wts'])
    wh = wh.at[:, 10:40].set(p['W_mean'])
    wh = wh.at[:, 40:70].set(p['W_fac'])
    wh = wh.at[:, 70:100].set(p['W_conc'])
    bh = jnp.zeros((128,), _f32)
    bh = bh.at[0:10].set(p['b_wts'])
    bh = bh.at[10:40].set(p['b_mean'])
    bh = bh.at[40:70].set(p['b_fac'])
    bh = bh.at[70:100].set(p['b_conc'])
    ang_p = jnp.pad(angles, ((0, 0), (0, 125)))
    out = _tc_head(x, ang_p, wh, row(bh))

    wts = out[:, 0:10]
    mean = out[:, 10:40].reshape(N, 3, MIX)
    factor = out[:, 40:70].reshape(N, 3, MIX)
    conc = out[:, 70:100].reshape(N, 3, MIX)
    return wts, mean, conc, factor


# speedup vs baseline: 6.8816x; 1.1138x over previous
import functools

import numpy as np
import jax
import jax.numpy as jnp
from jax import lax
from jax.experimental import pallas as pl
from jax.experimental.pallas import tpu as pltpu
from jax.experimental.pallas import tpu_sc as plsc

SIZE = 512
HEADS = 8
ATT = 64
HA = HEADS * ATT
MIX = 10
KERN = 16
MAXD = 20.0
EDGE_F = KERN + 9
HID = 1024
N = 4096
E = 65536

FW = 32
NC = 2
NS = 16
HHALF = HEADS // NC
DH = HHALF * ATT
QW = DH + HHALF * FW
KVW = 2 * DH
ACCW = DH + HHALF * FW
RT = 512
CH = 32
EPT = E // NS
NPT = N // NS
INV_SQRT_ATT = 1.0 / 8.0

_f32 = jnp.float32


def _ln(x, g, b):
    mu = jnp.mean(x, axis=-1, keepdims=True)
    var = jnp.mean((x - mu) ** 2, axis=-1, keepdims=True)
    return (x - mu) * lax.rsqrt(var + 1e-5) * g + b


def _pre_body(af_ref, wpre_ref, bpre_ref, wenc_ref, benc_ref, o_ref):
    h = jnp.dot(af_ref[...], wpre_ref[...], preferred_element_type=_f32) + bpre_ref[...]
    o_ref[...] = jnp.dot(h, wenc_ref[...], preferred_element_type=_f32) + benc_ref[...]


def _tc_pre(af_p, wpre_p, bpre, wenc, benc):
    return pl.pallas_call(
        _pre_body,
        grid=(N // RT,),
        in_specs=[
            pl.BlockSpec((RT, 128), lambda i: (i, 0)),
            pl.BlockSpec((128, SIZE), lambda i: (0, 0)),
            pl.BlockSpec((1, SIZE), lambda i: (0, 0)),
            pl.BlockSpec((SIZE, SIZE), lambda i: (0, 0)),
            pl.BlockSpec((1, SIZE), lambda i: (0, 0)),
        ],
        out_specs=pl.BlockSpec((RT, SIZE), lambda i: (i, 0)),
        out_shape=jax.ShapeDtypeStruct((N, SIZE), _f32),
    )(af_p, wpre_p, bpre, wenc, benc)


def _ef_body(prd_ref, prs_ref, g1_ref, g2_ref, g3_ref, o_ref):
    prd = prd_ref[:, 0:16]
    prs = prs_ref[:, 0:16]
    diff = prd - prs
    mask = lax.broadcasted_iota(jnp.int32, (1, 16), 1) < 3
    d2 = jnp.sum(jnp.where(mask, diff * diff, 0.0), axis=1, keepdims=True)
    d = jnp.sqrt(d2)
    centers = lax.broadcasted_iota(jnp.int32, (1, KERN), 1).astype(_f32) * (MAXD / (KERN - 1))
    sigma = MAXD / KERN
    t = (d - centers) / sigma
    rbf = jnp.exp(-(t * t))
    t1 = jnp.dot(prd, g1_ref[...], preferred_element_type=_f32)
    t2 = jnp.dot(prs, g2_ref[...], preferred_element_type=_f32)
    rel = jnp.dot(t1 * t2, g3_ref[...], preferred_element_type=_f32)
    r = rbf.shape[0]
    o_ref[:, 0:KERN] = rbf
    o_ref[:, KERN:KERN + 9] = rel[:, 0:9]
    o_ref[:, KERN + 9:KERN + 10] = jnp.ones((r, 1), _f32)
    o_ref[:, KERN + 10:FW] = jnp.zeros((r, FW - EDGE_F - 1), _f32)


def _tc_ef(prd, prs, g1, g2, g3):
    rt = 2048
    return pl.pallas_call(
        _ef_body,
        grid=(E // rt,),
        in_specs=[
            pl.BlockSpec((rt, 128), lambda i: (i, 0)),
            pl.BlockSpec((rt, 128), lambda i: (i, 0)),
            pl.BlockSpec((16, 32), lambda i: (0, 0)),
            pl.BlockSpec((16, 32), lambda i: (0, 0)),
            pl.BlockSpec((32, 16), lambda i: (0, 0)),
        ],
        out_specs=pl.BlockSpec((rt, FW), lambda i: (i, 0)),
        out_shape=jax.ShapeDtypeStruct((E, FW), _f32),
    )(prd, prs, g1, g2, g3)


def _blkA_body(x_ref, g_ref, b_ref, wq_ref, bq_ref, wk_ref, bk_ref,
               wv_ref, bv_ref, wbd_ref, qq_ref, kv_ref):
    h = _ln(x_ref[...], g_ref[...], b_ref[...])
    qs = (jnp.dot(h, wq_ref[...], preferred_element_type=_f32) + bq_ref[...]) * INV_SQRT_ATT
    kn = jnp.dot(h, wk_ref[...], preferred_element_type=_f32) + bk_ref[...]
    vn = jnp.dot(h, wv_ref[...], preferred_element_type=_f32) + bv_ref[...]
    qek = jnp.dot(qs, wbd_ref[...], preferred_element_type=_f32)
    qq_ref[0, :, 0:DH] = qs[:, 0:DH]
    qq_ref[0, :, DH:QW] = qek[:, 0:HHALF * FW]
    qq_ref[1, :, 0:DH] = qs[:, DH:HA]
    qq_ref[1, :, DH:QW] = qek[:, HHALF * FW:2 * HHALF * FW]
    kv_ref[0, :, 0:DH] = kn[:, 0:DH]
    kv_ref[0, :, DH:KVW] = vn[:, 0:DH]
    kv_ref[1, :, 0:DH] = kn[:, DH:HA]
    kv_ref[1, :, DH:KVW] = vn[:, DH:HA]


def _tc_blkA(x, g, b, wq, bq, wk, bk, wv, bv, wbd):
    full = lambda shape: pl.BlockSpec(shape, lambda i: tuple(0 for _ in shape))
    return pl.pallas_call(
        _blkA_body,
        grid=(N // RT,),
        in_specs=[
            pl.BlockSpec((RT, SIZE), lambda i: (i, 0)),
            full((1, SIZE)), full((1, SIZE)),
            full((SIZE, HA)), full((1, HA)),
            full((SIZE, HA)), full((1, HA)),
            full((SIZE, HA)), full((1, HA)),
            full((HA, HEADS * FW)),
        ],
        out_specs=[
            pl.BlockSpec((NC, RT, QW), lambda i: (0, i, 0)),
            pl.BlockSpec((NC, RT, KVW), lambda i: (0, i, 0)),
        ],
        out_shape=[
            jax.ShapeDtypeStruct((NC, N, QW), _f32),
            jax.ShapeDtypeStruct((NC, N, KVW), _f32),
        ],
    )(x, g, b, wq, bq, wk, bk, wv, bv, wbd)


def _blkB_body(x_ref, acc_ref, wev_ref, zsel_ref, wo_ref, bo_ref,
               g2_ref, b2_ref, w1_ref, b1_ref, w2_ref, bb2_ref, w3_ref, b3_ref,
               o_ref):
    numv = jnp.concatenate([acc_ref[0, :, 0:DH], acc_ref[1, :, 0:DH]], axis=1)
    pmat = jnp.concatenate([acc_ref[0, :, DH:ACCW], acc_ref[1, :, DH:ACCW]], axis=1)
    numc = jnp.dot(pmat, wev_ref[...], preferred_element_type=_f32)
    zfull = jnp.dot(pmat, zsel_ref[...], preferred_element_type=_f32)
    o = (numv + numc) / (zfull + 1e-9)
    x2 = x_ref[...] + jnp.dot(o, wo_ref[...], preferred_element_type=_f32) + bo_ref[...]
    h2 = _ln(x2, g2_ref[...], b2_ref[...])
    m1 = jnp.maximum(jnp.dot(h2, w1_ref[...], preferred_element_type=_f32) + b1_ref[...], 0.0)
    m2 = jnp.maximum(jnp.dot(m1, w2_ref[...], preferred_element_type=_f32) + bb2_ref[...], 0.0)
    o_ref[...] = x2 + jnp.dot(m2, w3_ref[...], preferred_element_type=_f32) + b3_ref[...]


def _tc_blkB(x, acc, wev, zsel, wo, bo, g2, b2, w1, b1, w2, bb2, w3, b3):
    full = lambda shape: pl.BlockSpec(shape, lambda i: tuple(0 for _ in shape))
    return pl.pallas_call(
        _blkB_body,
        grid=(N // RT,),
        in_specs=[
            pl.BlockSpec((RT, SIZE), lambda i: (i, 0)),
            pl.BlockSpec((NC, RT, ACCW), lambda i: (0, i, 0)),
            full((HEADS * FW, HA)), full((HEADS * FW, HA)),
            full((HA, SIZE)), full((1, SIZE)),
            full((1, SIZE)), full((1, SIZE)),
            full((SIZE, HID)), full((1, HID)),
            full((HID, HID)), full((1, HID)),
            full((HID, SIZE)), full((1, SIZE)),
        ],
        out_specs=pl.BlockSpec((RT, SIZE), lambda i: (i, 0)),
        out_shape=jax.ShapeDtypeStruct((N, SIZE), _f32),
    )(x, acc, wev, zsel, wo, bo, g2, b2, w1, b1, w2, bb2, w3, b3)


def _head_body(enc_ref, ang_ref, wh_ref, bh_ref, o_ref):
    t = jnp.dot(enc_ref[...], wh_ref[...], preferred_element_type=_f32) + bh_ref[...]
    lg = t[:, 0:MIX]
    m = jnp.max(lg, axis=1, keepdims=True)
    ex = jnp.exp(lg - m)
    wts = ex / jnp.sum(ex, axis=1, keepdims=True)
    a0 = ang_ref[:, 0:1]
    a1 = ang_ref[:, 1:2]
    f0 = t[:, 40:50]
    f1 = t[:, 50:60]
    f2 = t[:, 60:70]
    mean1 = t[:, 20:30] + f0 * a0
    mean2 = t[:, 30:40] + f1 * a0 + f2 * a1
    conc = 0.1 + 1000.0 / (1.0 + jnp.exp(-t[:, 70:100]))
    o_ref[:, 0:10] = wts
    o_ref[:, 10:20] = t[:, 10:20]
    o_ref[:, 20:30] = mean1
    o_ref[:, 30:40] = mean2
    o_ref[:, 40:70] = t[:, 40:70]
    o_ref[:, 70:100] = conc
    o_ref[:, 100:128] = jnp.zeros_like(t[:, 100:128])


def _tc_head(enc, ang_p, wh, bh):
    return pl.pallas_call(
        _head_body,
        grid=(N // RT,),
        in_specs=[
            pl.BlockSpec((RT, SIZE), lambda i: (i, 0)),
            pl.BlockSpec((RT, 128), lambda i: (i, 0)),
            pl.BlockSpec((SIZE, 128), lambda i: (0, 0)),
            pl.BlockSpec((1, 128), lambda i: (0, 0)),
        ],
        out_specs=pl.BlockSpec((RT, 128), lambda i: (i, 0)),
        out_shape=jax.ShapeDtypeStruct((N, 128), _f32),
    )(enc, ang_p, wh, bh)


CPR = 128


def _prgather_body(pr_hbm, dst_hbm, src_hbm, prd_hbm, prs_hbm, idx_v, buf_v, sem):
    c = lax.axis_index("c")
    s = lax.axis_index("s")
    wid = s * NC + c
    nper = E // (NC * NS)
    base0 = wid * nper

    def body(j, carry):
        base = base0 + j * CPR
        pltpu.sync_copy(dst_hbm.at[pl.ds(base, CPR)], idx_v)
        pltpu.async_copy(pr_hbm.at[idx_v], buf_v, sem).wait()
        pltpu.sync_copy(buf_v, prd_hbm.at[pl.ds(base, CPR)])
        pltpu.sync_copy(src_hbm.at[pl.ds(base, CPR)], idx_v)
        pltpu.async_copy(pr_hbm.at[idx_v], buf_v, sem).wait()
        pltpu.sync_copy(buf_v, prs_hbm.at[pl.ds(base, CPR)])
        return carry

    lax.fori_loop(0, nper // CPR, body, 0)


@functools.cache
def _get_sc_prgather():
    return pl.kernel(
        _prgather_body,
        out_type=(
            jax.ShapeDtypeStruct((E, 128), _f32),
            jax.ShapeDtypeStruct((E, 128), _f32),
        ),
        mesh=plsc.VectorSubcoreMesh(core_axis_name="c", subcore_axis_name="s"),
        scratch_types=[
            pltpu.VMEM((CPR,), jnp.int32),
            pltpu.VMEM((CPR, 128), _f32),
            pltpu.SemaphoreType.DMA,
        ],
    )


def _sc_prgather(pr, dst, src):
    return _get_sc_prgather()(pr, dst, src)


_GDN = lax.GatherDimensionNumbers(
    offset_dims=(), collapsed_slice_dims=(0,), start_index_map=(0,))


def _perm16(t, idx):
    return lax.gather(t, idx[:, None], dimension_numbers=_GDN,
                      slice_sizes=(1,),
                      mode=lax.GatherScatterMode.PROMISE_IN_BOUNDS)


def _allsum16(t):
    io = lax.iota(jnp.int32, 16)
    for k in (8, 4, 2, 1):
        t = t + _perm16(t, io ^ k)
    return t


SCN = 8
SCB = SCN * CH


def _edge_body(qq_hbm, kv_hbm, ef_hbm, dst_hbm, src_hbm, bounds_hbm, out_hbm,
               idx_dq, idx_sq, lloc, qq_b, kv_b, ef_b, acc_t, vmem_b,
               sdst, ssrc, sem0, sem1):
    c = lax.axis_index("c")
    s = lax.axis_index("s")
    cn = c * N
    io16 = lax.iota(jnp.int32, 16)

    zv = jnp.zeros((16,), _f32)

    def zrow(r, carry):
        for i in range(ACCW // 16):
            acc_t[pl.ds(r * ACCW + i * 16, 16)] = zv
        return carry

    lax.fori_loop(0, NPT, zrow, 0)

    pltpu.sync_copy(bounds_hbm, vmem_b)
    b_lo = vmem_b[pl.ds(0, 16)]
    b_hi = vmem_b[pl.ds(16, 16)]
    start_raw = jnp.sum(jnp.where(io16 == s, b_lo, 0))
    end_raw = (jnp.sum(jnp.where(io16 == s + 1, b_lo, 0)) +
               jnp.sum(jnp.where(io16 == s - 15, b_hi, 0)))
    start = (start_raw // CH) * CH
    nch = ((end_raw + CH - 1) // CH * CH - start) // CH

    def chunk(j, carry):
        base = start + j * CH

        @pl.when(lax.rem(j, SCN) == 0)
        def _():
            sb = base
            pltpu.sync_copy(dst_hbm.at[pl.ds(sb, SCB)], sdst)
            pltpu.sync_copy(src_hbm.at[pl.ds(sb, SCB)], ssrc)

        soff = lax.rem(j, SCN) * CH
        for g in range(CH // 16):
            dstv = sdst[pl.ds(soff + g * 16, 16)]
            lloc[pl.ds(g * 16, 16)] = dstv - s * NPT
            idx_dq[pl.ds(g * 16, 16)] = dstv + cn
            idx_sq[pl.ds(g * 16, 16)] = ssrc[pl.ds(soff + g * 16, 16)] + cn
        pltpu.sync_copy(ef_hbm.at[pl.ds(base * FW, CH * FW)], ef_b)
        cp0 = pltpu.async_copy(qq_hbm.at[idx_dq], qq_b, sem0)
        cp1 = pltpu.async_copy(kv_hbm.at[idx_sq], kv_b, sem1)
        cp0.wait()
        cp1.wait()

        def edge(e, ecarry):
            g16 = (e // 16) * 16
            loc = jnp.sum(jnp.where(io16 == e - g16, lloc[pl.ds(g16, 16)], 0))

            @pl.when((loc >= 0) & (loc < NPT))
            def _():
                rb = loc * ACCW
                ef0 = ef_b[pl.ds(e * FW, 16)]
                ef1 = ef_b[pl.ds(e * FW + 16, 16)]
                for h in range(HHALF):
                    qb = h * ATT
                    prod = (qq_b[e, pl.ds(qb, 16)] * kv_b[e, pl.ds(qb, 16)]
                            + qq_b[e, pl.ds(qb + 16, 16)] * kv_b[e, pl.ds(qb + 16, 16)]
                            + qq_b[e, pl.ds(qb + 32, 16)] * kv_b[e, pl.ds(qb + 32, 16)]
                            + qq_b[e, pl.ds(qb + 48, 16)] * kv_b[e, pl.ds(qb + 48, 16)])
                    p2 = (qq_b[e, pl.ds(DH + h * FW, 16)] * ef0
                          + qq_b[e, pl.ds(DH + h * FW + 16, 16)] * ef1)
                    exv = jnp.exp(_allsum16(prod + p2))
                    for i in range(ATT // 16):
                        plsc.addupdate(
                            acc_t.at[pl.ds(rb + qb + i * 16, 16)],
                            exv * kv_b[e, pl.ds(DH + qb + i * 16, 16)])
                    plsc.addupdate(acc_t.at[pl.ds(rb + DH + h * FW, 16)],
                                   exv * ef0)
                    plsc.addupdate(acc_t.at[pl.ds(rb + DH + h * FW + 16, 16)],
                                   exv * ef1)

            return ecarry

        lax.fori_loop(0, CH, edge, 0)
        return carry

    lax.fori_loop(0, nch, chunk, 0)
    pltpu.sync_copy(acc_t, out_hbm.at[pl.ds((cn + s * NPT) * ACCW, NPT * ACCW)])


@functools.cache
def _get_sc_edge():
    return pl.kernel(
        _edge_body,
        out_type=jax.ShapeDtypeStruct((NC * N * ACCW,), _f32),
        mesh=plsc.VectorSubcoreMesh(core_axis_name="c", subcore_axis_name="s"),
        compiler_params=pltpu.CompilerParams(needs_layout_passes=False),
        scratch_types=[
            pltpu.VMEM((CH,), jnp.int32),
            pltpu.VMEM((CH,), jnp.int32),
            pltpu.VMEM((CH,), jnp.int32),
            pltpu.VMEM((CH, QW), _f32),
            pltpu.VMEM((CH, KVW), _f32),
            pltpu.VMEM((CH * FW,), _f32),
            pltpu.VMEM((NPT * ACCW,), _f32),
            pltpu.VMEM((32,), jnp.int32),
            pltpu.VMEM((SCB,), jnp.int32),
            pltpu.VMEM((SCB,), jnp.int32),
            pltpu.SemaphoreType.DMA,
            pltpu.SemaphoreType.DMA,
        ],
    )


def _sc_edge(qq, kv, ef, dst, src, bounds):
    return _get_sc_edge()(qq, kv, ef, dst, src, bounds)


def _orient(tertiary):
    pos = tertiary[:, 1]
    nxt = jnp.roll(pos, -1, axis=0)
    prv = jnp.roll(pos, 1, axis=0)
    a = nxt - pos
    a = a / (jnp.linalg.norm(a, axis=-1, keepdims=True) + 1e-8)
    cc = pos - prv
    cc = cc / (jnp.linalg.norm(cc, axis=-1, keepdims=True) + 1e-8)
    nvec = jnp.cross(a, cc)
    nvec = nvec / (jnp.linalg.norm(nvec, axis=-1, keepdims=True) + 1e-8)
    m = jnp.cross(nvec, a)
    return jnp.stack([a, m, nvec], axis=1)


def _sel_mats():
    g1 = np.zeros((16, 32), np.float32)
    g2 = np.zeros((16, 32), np.float32)
    g3 = np.zeros((32, 16), np.float32)
    for i in range(3):
        for j in range(3):
            for k in range(3):
                b = 9 * i + 3 * j + k
                g1[3 + 3 * i + j, b] = 1.0
                g2[3 + 3 * i + k, b] = 1.0
                g3[b, 3 * j + k] = 1.0
    return g1, g2, g3[:, :16]


_G1, _G2, _G3 = _sel_mats()


def _block_weights(blk):
    wek = blk['Wek']
    bek = blk['bek']
    wev = blk['Wev']
    wbd = jnp.zeros((HA, HEADS * FW), _f32)
    wevbd = jnp.zeros((HEADS * FW, HA), _f32)
    zsel = jnp.zeros((HEADS * FW, HA), _f32)
    for h in range(HEADS):
        cs = slice(h * ATT, (h + 1) * ATT)
        wbd = wbd.at[cs, h * FW:h * FW + EDGE_F].set(wek[:, cs].T)
        wbd = wbd.at[cs, h * FW + EDGE_F].set(bek[cs])
        wevbd = wevbd.at[h * FW:h * FW + EDGE_F, cs].set(wev[:, cs])
        zsel = zsel.at[h * FW + EDGE_F, cs].set(1.0)
    return wbd, wevbd, zsel


def kernel(angles, tertiary, params, edge_index, subgraph):
    p = params
    row = lambda v: v.reshape(1, -1)

    perm = jnp.argsort(edge_index[0])
    dst = edge_index[0][perm]
    src = edge_index[1][perm]
    boundaries = jnp.arange(0, N + 1, NPT, dtype=jnp.int32)
    bounds = jnp.searchsorted(dst, boundaries).astype(jnp.int32)
    bounds_p = jnp.zeros((32,), jnp.int32).at[0:NS + 1].set(bounds)
    dst_p = jnp.concatenate([dst, jnp.zeros((SCB,), jnp.int32)])
    src_p = jnp.concatenate([src, jnp.zeros((SCB,), jnp.int32)])

    prev = jnp.roll(angles, 1, axis=0).at[0].set(0.0)
    afeat = jnp.concatenate([jnp.sin(prev), jnp.cos(prev)], axis=1)
    af_p = jnp.pad(afeat, ((0, 0), (0, 122)))
    wpre_p = jnp.pad(p['W_pre'], ((0, 122), (0, 0)))

    pos = tertiary[:, 1]
    rot = _orient(tertiary)
    pr = jnp.concatenate([pos, rot.reshape(N, 9), jnp.zeros((N, 116), _f32)], axis=1)

    x = _tc_pre(af_p, wpre_p, row(p['b_pre']), p['W_enc'], row(p['b_enc']))

    prd, prs = _sc_prgather(pr, dst, src)
    ef = _tc_ef(prd, prs, _G1, _G2, _G3)
    ef_flat = ef.reshape(-1)

    for blk in p['blocks']:
        wbd, wevbd, zsel = _block_weights(blk)
        qq, kv = _tc_blkA(
            x, row(blk['ln1_g']), row(blk['ln1_b']),
            blk['Wq'], row(blk['bq']),
            blk['Wk'], row(blk['bk']),
            blk['Wv'], row(blk['bv'] + blk['bev']),
            wbd)
        acc = _sc_edge(qq.reshape(NC * N, QW), kv.reshape(NC * N, KVW),
                       ef_flat, dst_p, src_p, bounds_p)
        x = _tc_blkB(
            x, acc.reshape(NC, N, ACCW), wevbd, zsel,
            blk['Wo'], row(blk['bo']),
            row(blk['ln2_g']), row(blk['ln2_b']),
            blk['W1'], row(blk['b1']),
            blk['W2'], row(blk['b2']),
            blk['W3'], row(blk['b3']))

    wh = jnp.zeros((SIZE, 128), _f32)
    wh = wh.at[:, 0:10].set(p['W_wts'])
    wh = wh.at[:, 10:40].set(p['W_mean'])
    wh = wh.at[:, 40:70].set(p['W_fac'])
    wh = wh.at[:, 70:100].set(p['W_conc'])
    bh = jnp.zeros((128,), _f32)
    bh = bh.at[0:10].set(p['b_wts'])
    bh = bh.at[10:40].set(p['b_mean'])
    bh = bh.at[40:70].set(p['b_fac'])
    bh = bh.at[70:100].set(p['b_conc'])
    ang_p = jnp.pad(angles, ((0, 0), (0, 125)))
    out = _tc_head(x, ang_p, wh, row(bh))

    wts = out[:, 0:10]
    mean = out[:, 10:40].reshape(N, 3, MIX)
    factor = out[:, 40:70].reshape(N, 3, MIX)
    conc = out[:, 70:100].reshape(N, 3, MIX)
    return wts, mean, conc, factor


# speedup vs baseline: 7.3783x; 1.0722x over previous
import functools

import numpy as np
import jax
import jax.numpy as jnp
from jax import lax
from jax.experimental import pallas as pl
from jax.experimental.pallas import tpu as pltpu
from jax.experimental.pallas import tpu_sc as plsc

SIZE = 512
HEADS = 8
ATT = 64
HA = HEADS * ATT
MIX = 10
KERN = 16
MAXD = 20.0
EDGE_F = KERN + 9
HID = 1024
N = 4096
E = 65536

FW = 32
NC = 2
NS = 16
HHALF = HEADS // NC
DH = HHALF * ATT
QW = DH + HHALF * FW
KVW = 2 * DH
ACCW = DH + HHALF * FW
RT = 512
CH = 32
EPT = E // NS
NPT = N // NS
INV_SQRT_ATT = 1.0 / 8.0

_f32 = jnp.float32


def _ln(x, g, b):
    mu = jnp.mean(x, axis=-1, keepdims=True)
    var = jnp.mean((x - mu) ** 2, axis=-1, keepdims=True)
    return (x - mu) * lax.rsqrt(var + 1e-5) * g + b


def _pre_body(af_ref, wpre_ref, bpre_ref, wenc_ref, benc_ref, o_ref):
    h = jnp.dot(af_ref[...], wpre_ref[...], preferred_element_type=_f32) + bpre_ref[...]
    o_ref[...] = jnp.dot(h, wenc_ref[...], preferred_element_type=_f32) + benc_ref[...]


def _tc_pre(af_p, wpre_p, bpre, wenc, benc):
    return pl.pallas_call(
        _pre_body,
        grid=(N // RT,),
        in_specs=[
            pl.BlockSpec((RT, 128), lambda i: (i, 0)),
            pl.BlockSpec((128, SIZE), lambda i: (0, 0)),
            pl.BlockSpec((1, SIZE), lambda i: (0, 0)),
            pl.BlockSpec((SIZE, SIZE), lambda i: (0, 0)),
            pl.BlockSpec((1, SIZE), lambda i: (0, 0)),
        ],
        out_specs=pl.BlockSpec((RT, SIZE), lambda i: (i, 0)),
        out_shape=jax.ShapeDtypeStruct((N, SIZE), _f32),
    )(af_p, wpre_p, bpre, wenc, benc)


def _ef_body(prd_ref, prs_ref, g1_ref, g2_ref, g3_ref, o_ref):
    prd = prd_ref[:, 0:16]
    prs = prs_ref[:, 0:16]
    diff = prd - prs
    mask = lax.broadcasted_iota(jnp.int32, (1, 16), 1) < 3
    d2 = jnp.sum(jnp.where(mask, diff * diff, 0.0), axis=1, keepdims=True)
    d = jnp.sqrt(d2)
    centers = lax.broadcasted_iota(jnp.int32, (1, KERN), 1).astype(_f32) * (MAXD / (KERN - 1))
    sigma = MAXD / KERN
    t = (d - centers) / sigma
    rbf = jnp.exp(-(t * t))
    t1 = jnp.dot(prd, g1_ref[...], preferred_element_type=_f32)
    t2 = jnp.dot(prs, g2_ref[...], preferred_element_type=_f32)
    rel = jnp.dot(t1 * t2, g3_ref[...], preferred_element_type=_f32)
    r = rbf.shape[0]
    o_ref[:, 0:KERN] = rbf
    o_ref[:, KERN:KERN + 9] = rel[:, 0:9]
    o_ref[:, KERN + 9:KERN + 10] = jnp.ones((r, 1), _f32)
    o_ref[:, KERN + 10:FW] = jnp.zeros((r, FW - EDGE_F - 1), _f32)


def _tc_ef(prd, prs, g1, g2, g3):
    rt = 2048
    return pl.pallas_call(
        _ef_body,
        grid=(E // rt,),
        in_specs=[
            pl.BlockSpec((rt, 128), lambda i: (i, 0)),
            pl.BlockSpec((rt, 128), lambda i: (i, 0)),
            pl.BlockSpec((16, 32), lambda i: (0, 0)),
            pl.BlockSpec((16, 32), lambda i: (0, 0)),
            pl.BlockSpec((32, 16), lambda i: (0, 0)),
        ],
        out_specs=pl.BlockSpec((rt, FW), lambda i: (i, 0)),
        out_shape=jax.ShapeDtypeStruct((E, FW), _f32),
    )(prd, prs, g1, g2, g3)


def _blkA_body(x_ref, g_ref, b_ref, wq_ref, bq_ref, wk_ref, bk_ref,
               wv_ref, bv_ref, wbd_ref, qq_ref, kv_ref):
    h = _ln(x_ref[...], g_ref[...], b_ref[...])
    qs = (jnp.dot(h, wq_ref[...], preferred_element_type=_f32) + bq_ref[...]) * INV_SQRT_ATT
    kn = jnp.dot(h, wk_ref[...], preferred_element_type=_f32) + bk_ref[...]
    vn = jnp.dot(h, wv_ref[...], preferred_element_type=_f32) + bv_ref[...]
    qek = jnp.dot(qs, wbd_ref[...], preferred_element_type=_f32)
    qq_ref[0, :, 0:DH] = qs[:, 0:DH]
    qq_ref[0, :, DH:QW] = qek[:, 0:HHALF * FW]
    qq_ref[1, :, 0:DH] = qs[:, DH:HA]
    qq_ref[1, :, DH:QW] = qek[:, HHALF * FW:2 * HHALF * FW]
    kv_ref[0, :, 0:DH] = kn[:, 0:DH]
    kv_ref[0, :, DH:KVW] = vn[:, 0:DH]
    kv_ref[1, :, 0:DH] = kn[:, DH:HA]
    kv_ref[1, :, DH:KVW] = vn[:, DH:HA]


def _tc_blkA(x, g, b, wq, bq, wk, bk, wv, bv, wbd):
    full = lambda shape: pl.BlockSpec(shape, lambda i: tuple(0 for _ in shape))
    return pl.pallas_call(
        _blkA_body,
        grid=(N // RT,),
        in_specs=[
            pl.BlockSpec((RT, SIZE), lambda i: (i, 0)),
            full((1, SIZE)), full((1, SIZE)),
            full((SIZE, HA)), full((1, HA)),
            full((SIZE, HA)), full((1, HA)),
            full((SIZE, HA)), full((1, HA)),
            full((HA, HEADS * FW)),
        ],
        out_specs=[
            pl.BlockSpec((NC, RT, QW), lambda i: (0, i, 0)),
            pl.BlockSpec((NC, RT, KVW), lambda i: (0, i, 0)),
        ],
        out_shape=[
            jax.ShapeDtypeStruct((NC, N, QW), _f32),
            jax.ShapeDtypeStruct((NC, N, KVW), _f32),
        ],
    )(x, g, b, wq, bq, wk, bk, wv, bv, wbd)


def _blkB_body(x_ref, acc_ref, wev_ref, zsel_ref, wo_ref, bo_ref,
               g2_ref, b2_ref, w1_ref, b1_ref, w2_ref, bb2_ref, w3_ref, b3_ref,
               o_ref):
    numv = jnp.concatenate([acc_ref[0, :, 0:DH], acc_ref[1, :, 0:DH]], axis=1)
    pmat = jnp.concatenate([acc_ref[0, :, DH:ACCW], acc_ref[1, :, DH:ACCW]], axis=1)
    numc = jnp.dot(pmat, wev_ref[...], preferred_element_type=_f32)
    zfull = jnp.dot(pmat, zsel_ref[...], preferred_element_type=_f32)
    o = (numv + numc) / (zfull + 1e-9)
    x2 = x_ref[...] + jnp.dot(o, wo_ref[...], preferred_element_type=_f32) + bo_ref[...]
    h2 = _ln(x2, g2_ref[...], b2_ref[...])
    m1 = jnp.maximum(jnp.dot(h2, w1_ref[...], preferred_element_type=_f32) + b1_ref[...], 0.0)
    m2 = jnp.maximum(jnp.dot(m1, w2_ref[...], preferred_element_type=_f32) + bb2_ref[...], 0.0)
    o_ref[...] = x2 + jnp.dot(m2, w3_ref[...], preferred_element_type=_f32) + b3_ref[...]


def _tc_blkB(x, acc, wev, zsel, wo, bo, g2, b2, w1, b1, w2, bb2, w3, b3):
    full = lambda shape: pl.BlockSpec(shape, lambda i: tuple(0 for _ in shape))
    return pl.pallas_call(
        _blkB_body,
        grid=(N // RT,),
        in_specs=[
            pl.BlockSpec((RT, SIZE), lambda i: (i, 0)),
            pl.BlockSpec((NC, RT, ACCW), lambda i: (0, i, 0)),
            full((HEADS * FW, HA)), full((HEADS * FW, HA)),
            full((HA, SIZE)), full((1, SIZE)),
            full((1, SIZE)), full((1, SIZE)),
            full((SIZE, HID)), full((1, HID)),
            full((HID, HID)), full((1, HID)),
            full((HID, SIZE)), full((1, SIZE)),
        ],
        out_specs=pl.BlockSpec((RT, SIZE), lambda i: (i, 0)),
        out_shape=jax.ShapeDtypeStruct((N, SIZE), _f32),
    )(x, acc, wev, zsel, wo, bo, g2, b2, w1, b1, w2, bb2, w3, b3)


def _head_body(enc_ref, ang_ref, wh_ref, bh_ref, o_ref):
    t = jnp.dot(enc_ref[...], wh_ref[...], preferred_element_type=_f32) + bh_ref[...]
    lg = t[:, 0:MIX]
    m = jnp.max(lg, axis=1, keepdims=True)
    ex = jnp.exp(lg - m)
    wts = ex / jnp.sum(ex, axis=1, keepdims=True)
    a0 = ang_ref[:, 0:1]
    a1 = ang_ref[:, 1:2]
    f0 = t[:, 40:50]
    f1 = t[:, 50:60]
    f2 = t[:, 60:70]
    mean1 = t[:, 20:30] + f0 * a0
    mean2 = t[:, 30:40] + f1 * a0 + f2 * a1
    conc = 0.1 + 1000.0 / (1.0 + jnp.exp(-t[:, 70:100]))
    o_ref[:, 0:10] = wts
    o_ref[:, 10:20] = t[:, 10:20]
    o_ref[:, 20:30] = mean1
    o_ref[:, 30:40] = mean2
    o_ref[:, 40:70] = t[:, 40:70]
    o_ref[:, 70:100] = conc
    o_ref[:, 100:128] = jnp.zeros_like(t[:, 100:128])


def _tc_head(enc, ang_p, wh, bh):
    return pl.pallas_call(
        _head_body,
        grid=(N // RT,),
        in_specs=[
            pl.BlockSpec((RT, SIZE), lambda i: (i, 0)),
            pl.BlockSpec((RT, 128), lambda i: (i, 0)),
            pl.BlockSpec((SIZE, 128), lambda i: (0, 0)),
            pl.BlockSpec((1, 128), lambda i: (0, 0)),
        ],
        out_specs=pl.BlockSpec((RT, 128), lambda i: (i, 0)),
        out_shape=jax.ShapeDtypeStruct((N, 128), _f32),
    )(enc, ang_p, wh, bh)


CPR = 128


def _prgather_body(pr_hbm, dst_hbm, src_hbm, prd_hbm, prs_hbm, idx_v, buf_v, sem):
    c = lax.axis_index("c")
    s = lax.axis_index("s")
    wid = s * NC + c
    nper = E // (NC * NS)
    base0 = wid * nper

    def body(j, carry):
        base = base0 + j * CPR
        pltpu.sync_copy(dst_hbm.at[pl.ds(base, CPR)], idx_v)
        pltpu.async_copy(pr_hbm.at[idx_v], buf_v, sem).wait()
        pltpu.sync_copy(buf_v, prd_hbm.at[pl.ds(base, CPR)])
        pltpu.sync_copy(src_hbm.at[pl.ds(base, CPR)], idx_v)
        pltpu.async_copy(pr_hbm.at[idx_v], buf_v, sem).wait()
        pltpu.sync_copy(buf_v, prs_hbm.at[pl.ds(base, CPR)])
        return carry

    lax.fori_loop(0, nper // CPR, body, 0)


@functools.cache
def _get_sc_prgather():
    return pl.kernel(
        _prgather_body,
        out_type=(
            jax.ShapeDtypeStruct((E, 128), _f32),
            jax.ShapeDtypeStruct((E, 128), _f32),
        ),
        mesh=plsc.VectorSubcoreMesh(core_axis_name="c", subcore_axis_name="s"),
        scratch_types=[
            pltpu.VMEM((CPR,), jnp.int32),
            pltpu.VMEM((CPR, 128), _f32),
            pltpu.SemaphoreType.DMA,
        ],
    )


def _sc_prgather(pr, dst, src):
    return _get_sc_prgather()(pr, dst, src)


_GDN = lax.GatherDimensionNumbers(
    offset_dims=(), collapsed_slice_dims=(0,), start_index_map=(0,))


def _perm16(t, idx):
    return lax.gather(t, idx[:, None], dimension_numbers=_GDN,
                      slice_sizes=(1,),
                      mode=lax.GatherScatterMode.PROMISE_IN_BOUNDS)


def _allsum16(t):
    io = lax.iota(jnp.int32, 16)
    for k in (8, 4, 2, 1):
        t = t + _perm16(t, io ^ k)
    return t


SCN = 8
SCB = SCN * CH


def _edge_body(qq_hbm, kv_hbm, ef_hbm, dst_hbm, src_hbm, bounds_hbm, out_hbm,
               idx_dq, idx_sq, lloc, qq_b, kv_b, ef_b, acc_t, vmem_b,
               sdst, ssrc, sem0, sem1, sem2):
    c = lax.axis_index("c")
    s = lax.axis_index("s")
    cn = c * N
    io16 = lax.iota(jnp.int32, 16)

    zv = jnp.zeros((16,), _f32)

    def zrow(r, carry):
        for i in range(ACCW // 16):
            acc_t[pl.ds(r * ACCW + i * 16, 16)] = zv
        return carry

    lax.fori_loop(0, NPT, zrow, 0)

    pltpu.sync_copy(bounds_hbm, vmem_b)
    b_lo = vmem_b[pl.ds(0, 16)]
    b_hi = vmem_b[pl.ds(16, 16)]
    start_raw = jnp.sum(jnp.where(io16 == s, b_lo, 0))
    end_raw = (jnp.sum(jnp.where(io16 == s + 1, b_lo, 0)) +
               jnp.sum(jnp.where(io16 == s - 15, b_hi, 0)))
    start = (start_raw // CH) * CH
    nch = ((end_raw + CH - 1) // CH * CH - start) // CH

    def chunk(j, carry):
        base = start + j * CH

        @pl.when(lax.rem(j, SCN) == 0)
        def _():
            sb = base
            pltpu.sync_copy(dst_hbm.at[pl.ds(sb, SCB)], sdst)
            pltpu.sync_copy(src_hbm.at[pl.ds(sb, SCB)], ssrc)

        soff = lax.rem(j, SCN) * CH
        for g in range(CH // 16):
            dstv = sdst[pl.ds(soff + g * 16, 16)]
            lloc[pl.ds(g * 16, 16)] = dstv - s * NPT
            idx_dq[pl.ds(g * 16, 16)] = dstv + cn
            idx_sq[pl.ds(g * 16, 16)] = ssrc[pl.ds(soff + g * 16, 16)] + cn
        cp0 = pltpu.async_copy(qq_hbm.at[idx_dq], qq_b, sem0)
        cp1 = pltpu.async_copy(kv_hbm.at[idx_sq], kv_b, sem1)
        cp2 = pltpu.async_copy(ef_hbm.at[pl.ds(base * FW, CH * FW)], ef_b, sem2)
        cp0.wait()
        cp1.wait()
        cp2.wait()

        def edge(e, ecarry):
            g16 = (e // 16) * 16
            loc = jnp.sum(jnp.where(io16 == e - g16, lloc[pl.ds(g16, 16)], 0))

            @pl.when((loc >= 0) & (loc < NPT))
            def _():
                rb = loc * ACCW
                ef0 = ef_b[pl.ds(e * FW, 16)]
                ef1 = ef_b[pl.ds(e * FW + 16, 16)]
                for h in range(HHALF):
                    qb = h * ATT
                    prod = (qq_b[e, pl.ds(qb, 16)] * kv_b[e, pl.ds(qb, 16)]
                            + qq_b[e, pl.ds(qb + 16, 16)] * kv_b[e, pl.ds(qb + 16, 16)]
                            + qq_b[e, pl.ds(qb + 32, 16)] * kv_b[e, pl.ds(qb + 32, 16)]
                            + qq_b[e, pl.ds(qb + 48, 16)] * kv_b[e, pl.ds(qb + 48, 16)])
                    p2 = (qq_b[e, pl.ds(DH + h * FW, 16)] * ef0
                          + qq_b[e, pl.ds(DH + h * FW + 16, 16)] * ef1)
                    exv = jnp.exp(jnp.full((16,), jnp.sum(prod + p2), _f32))
                    for i in range(ATT // 16):
                        plsc.addupdate(
                            acc_t.at[pl.ds(rb + qb + i * 16, 16)],
                            exv * kv_b[e, pl.ds(DH + qb + i * 16, 16)])
                    plsc.addupdate(acc_t.at[pl.ds(rb + DH + h * FW, 16)],
                                   exv * ef0)
                    plsc.addupdate(acc_t.at[pl.ds(rb + DH + h * FW + 16, 16)],
                                   exv * ef1)

            return ecarry

        lax.fori_loop(0, CH, edge, 0)
        return carry

    lax.fori_loop(0, nch, chunk, 0)
    pltpu.sync_copy(acc_t, out_hbm.at[pl.ds((cn + s * NPT) * ACCW, NPT * ACCW)])


@functools.cache
def _get_sc_edge():
    return pl.kernel(
        _edge_body,
        out_type=jax.ShapeDtypeStruct((NC * N * ACCW,), _f32),
        mesh=plsc.VectorSubcoreMesh(core_axis_name="c", subcore_axis_name="s"),
        compiler_params=pltpu.CompilerParams(needs_layout_passes=False),
        scratch_types=[
            pltpu.VMEM((CH,), jnp.int32),
            pltpu.VMEM((CH,), jnp.int32),
            pltpu.VMEM((CH,), jnp.int32),
            pltpu.VMEM((CH, QW), _f32),
            pltpu.VMEM((CH, KVW), _f32),
            pltpu.VMEM((CH * FW,), _f32),
            pltpu.VMEM((NPT * ACCW,), _f32),
            pltpu.VMEM((32,), jnp.int32),
            pltpu.VMEM((SCB,), jnp.int32),
            pltpu.VMEM((SCB,), jnp.int32),
            pltpu.SemaphoreType.DMA,
            pltpu.SemaphoreType.DMA,
            pltpu.SemaphoreType.DMA,
        ],
    )


def _sc_edge(qq, kv, ef, dst, src, bounds):
    return _get_sc_edge()(qq, kv, ef, dst, src, bounds)


def _orient(tertiary):
    pos = tertiary[:, 1]
    nxt = jnp.roll(pos, -1, axis=0)
    prv = jnp.roll(pos, 1, axis=0)
    a = nxt - pos
    a = a / (jnp.linalg.norm(a, axis=-1, keepdims=True) + 1e-8)
    cc = pos - prv
    cc = cc / (jnp.linalg.norm(cc, axis=-1, keepdims=True) + 1e-8)
    nvec = jnp.cross(a, cc)
    nvec = nvec / (jnp.linalg.norm(nvec, axis=-1, keepdims=True) + 1e-8)
    m = jnp.cross(nvec, a)
    return jnp.stack([a, m, nvec], axis=1)


def _sel_mats():
    g1 = np.zeros((16, 32), np.float32)
    g2 = np.zeros((16, 32), np.float32)
    g3 = np.zeros((32, 16), np.float32)
    for i in range(3):
        for j in range(3):
            for k in range(3):
                b = 9 * i + 3 * j + k
                g1[3 + 3 * i + j, b] = 1.0
                g2[3 + 3 * i + k, b] = 1.0
                g3[b, 3 * j + k] = 1.0
    return g1, g2, g3[:, :16]


_G1, _G2, _G3 = _sel_mats()


def _block_weights(blk):
    wek = blk['Wek']
    bek = blk['bek']
    wev = blk['Wev']
    wbd = jnp.zeros((HA, HEADS * FW), _f32)
    wevbd = jnp.zeros((HEADS * FW, HA), _f32)
    zsel = jnp.zeros((HEADS * FW, HA), _f32)
    for h in range(HEADS):
        cs = slice(h * ATT, (h + 1) * ATT)
        wbd = wbd.at[cs, h * FW:h * FW + EDGE_F].set(wek[:, cs].T)
        wbd = wbd.at[cs, h * FW + EDGE_F].set(bek[cs])
        wevbd = wevbd.at[h * FW:h * FW + EDGE_F, cs].set(wev[:, cs])
        zsel = zsel.at[h * FW + EDGE_F, cs].set(1.0)
    return wbd, wevbd, zsel


def kernel(angles, tertiary, params, edge_index, subgraph):
    p = params
    row = lambda v: v.reshape(1, -1)

    perm = jnp.argsort(edge_index[0])
    dst = edge_index[0][perm]
    src = edge_index[1][perm]
    boundaries = jnp.arange(0, N + 1, NPT, dtype=jnp.int32)
    bounds = jnp.searchsorted(dst, boundaries).astype(jnp.int32)
    bounds_p = jnp.zeros((32,), jnp.int32).at[0:NS + 1].set(bounds)
    dst_p = jnp.concatenate([dst, jnp.zeros((SCB,), jnp.int32)])
    src_p = jnp.concatenate([src, jnp.zeros((SCB,), jnp.int32)])

    prev = jnp.roll(angles, 1, axis=0).at[0].set(0.0)
    afeat = jnp.concatenate([jnp.sin(prev), jnp.cos(prev)], axis=1)
    af_p = jnp.pad(afeat, ((0, 0), (0, 122)))
    wpre_p = jnp.pad(p['W_pre'], ((0, 122), (0, 0)))

    pos = tertiary[:, 1]
    rot = _orient(tertiary)
    pr = jnp.concatenate([pos, rot.reshape(N, 9), jnp.zeros((N, 116), _f32)], axis=1)

    x = _tc_pre(af_p, wpre_p, row(p['b_pre']), p['W_enc'], row(p['b_enc']))

    prd, prs = _sc_prgather(pr, dst, src)
    ef = _tc_ef(prd, prs, _G1, _G2, _G3)
    ef_flat = ef.reshape(-1)

    for blk in p['blocks']:
        wbd, wevbd, zsel = _block_weights(blk)
        qq, kv = _tc_blkA(
            x, row(blk['ln1_g']), row(blk['ln1_b']),
            blk['Wq'], row(blk['bq']),
            blk['Wk'], row(blk['bk']),
            blk['Wv'], row(blk['bv'] + blk['bev']),
            wbd)
        acc = _sc_edge(qq.reshape(NC * N, QW), kv.reshape(NC * N, KVW),
                       ef_flat, dst_p, src_p, bounds_p)
        x = _tc_blkB(
            x, acc.reshape(NC, N, ACCW), wevbd, zsel,
            blk['Wo'], row(blk['bo']),
            row(blk['ln2_g']), row(blk['ln2_b']),
            blk['W1'], row(blk['b1']),
            blk['W2'], row(blk['b2']),
            blk['W3'], row(blk['b3']))

    wh = jnp.zeros((SIZE, 128), _f32)
    wh = wh.at[:, 0:10].set(p['W_wts'])
    wh = wh.at[:, 10:40].set(p['W_mean'])
    wh = wh.at[:, 40:70].set(p['W_fac'])
    wh = wh.at[:, 70:100].set(p['W_conc'])
    bh = jnp.zeros((128,), _f32)
    bh = bh.at[0:10].set(p['b_wts'])
    bh = bh.at[10:40].set(p['b_mean'])
    bh = bh.at[40:70].set(p['b_fac'])
    bh = bh.at[70:100].set(p['b_conc'])
    ang_p = jnp.pad(angles, ((0, 0), (0, 125)))
    out = _tc_head(x, ang_p, wh, row(bh))

    wts = out[:, 0:10]
    mean = out[:, 10:40].reshape(N, 3, MIX)
    factor = out[:, 40:70].reshape(N, 3, MIX)
    conc = out[:, 70:100].reshape(N, 3, MIX)
    return wts, mean, conc, factor
